# baseline (speedup 1.0000x reference)
import functools

import jax
import jax.numpy as jnp
from jax import lax
from jax.experimental import pallas as pl
from jax.experimental.pallas import tpu as pltpu
from jax.experimental.pallas import tpu_sc as plsc

_LENGTH = 4096
_N_EMBD = 1024
_NC = 2
_NS = 16
_NW = _NC * _NS
_ROWS_PER_W = _LENGTH // _NW
_CHUNK = 16
_NCHUNK = _ROWS_PER_W // _CHUNK
_NBUF = 7


def _make_sc_copy():
    mesh = plsc.VectorSubcoreMesh(core_axis_name="c", subcore_axis_name="s")

    @functools.partial(
        pl.kernel,
        mesh=mesh,
        out_type=jax.ShapeDtypeStruct((_LENGTH, _N_EMBD), jnp.float32),
        scratch_types=(
            [pltpu.VMEM((_CHUNK, _N_EMBD), jnp.float32) for _ in range(_NBUF)]
            + [pltpu.SemaphoreType.DMA for _ in range(2 * _NBUF)]
        ),
    )
    def sc_copy(table_hbm, out_hbm, *scratch):
        bufs = scratch[:_NBUF]
        in_sems = scratch[_NBUF:2 * _NBUF]
        out_sems = scratch[2 * _NBUF:]
        wid = lax.axis_index("s") * _NC + lax.axis_index("c")
        base = wid * _ROWS_PER_W

        def start_in(i):
            cp = pltpu.make_async_copy(
                table_hbm.at[pl.ds(base + i * _CHUNK, _CHUNK)],
                bufs[i % _NBUF],
                in_sems[i % _NBUF],
            )
            cp.start()
            return cp

        def start_out(i):
            cp = pltpu.make_async_copy(
                bufs[i % _NBUF],
                out_hbm.at[pl.ds(base + i * _CHUNK, _CHUNK)],
                out_sems[i % _NBUF],
            )
            cp.start()
            return cp

        in_cps = [None] * _NCHUNK
        out_cps = [None] * _NCHUNK
        for b in range(_NBUF):
            in_cps[b] = start_in(b)
        for i in range(_NCHUNK):
            in_cps[i].wait()
            out_cps[i] = start_out(i)
            j = i + _NBUF
            if j < _NCHUNK:
                out_cps[i].wait()
                in_cps[j] = start_in(j)
        for i in range(_NCHUNK - _NBUF, _NCHUNK):
            out_cps[i].wait()

    return sc_copy


_sc_copy = _make_sc_copy()


@jax.jit
def kernel(pos_embd_weight):
    return _sc_copy(pos_embd_weight)[None]


def _tc_copy(x):
    def body(in_ref, out_ref, sem):
        pltpu.make_async_copy(in_ref, out_ref, sem).start()
        pltpu.make_async_copy(in_ref, out_ref, sem).wait()

    return pl.pallas_call(
        body,
        out_shape=jax.ShapeDtypeStruct((_LENGTH, _N_EMBD), jnp.float32),
        in_specs=[pl.BlockSpec(memory_space=pl.ANY)],
        out_specs=pl.BlockSpec(memory_space=pl.ANY),
        scratch_shapes=[pltpu.SemaphoreType.DMA],
    )(x)


@jax.jit
def _kernel_tc(pos_embd_weight):
    return _tc_copy(pos_embd_weight)[None]

kernel = _kernel_tc


# speedup vs baseline: 29.0920x; 29.0920x over previous
import functools

import jax
import jax.numpy as jnp
from jax import lax
from jax.experimental import pallas as pl
from jax.experimental.pallas import tpu as pltpu
from jax.experimental.pallas import tpu_sc as plsc

_LENGTH = 4096
_N_EMBD = 1024
_NC = 2
_NS = 16
_NW = _NC * _NS
_ROWS_PER_W = _LENGTH // _NW
_CHUNK = 16
_NCHUNK = _ROWS_PER_W // _CHUNK
_NBUF = 7


def _make_sc_copy():
    mesh = plsc.VectorSubcoreMesh(core_axis_name="c", subcore_axis_name="s")

    @functools.partial(
        pl.kernel,
        mesh=mesh,
        out_type=jax.ShapeDtypeStruct((_LENGTH, _N_EMBD), jnp.float32),
        scratch_types=(
            [pltpu.VMEM((_CHUNK, _N_EMBD), jnp.float32) for _ in range(_NBUF)]
            + [pltpu.SemaphoreType.DMA for _ in range(2 * _NBUF)]
        ),
    )
    def sc_copy(table_hbm, out_hbm, *scratch):
        bufs = scratch[:_NBUF]
        in_sems = scratch[_NBUF:2 * _NBUF]
        out_sems = scratch[2 * _NBUF:]
        wid = lax.axis_index("s") * _NC + lax.axis_index("c")
        base = wid * _ROWS_PER_W

        def start_in(i):
            cp = pltpu.make_async_copy(
                table_hbm.at[pl.ds(base + i * _CHUNK, _CHUNK)],
                bufs[i % _NBUF],
                in_sems[i % _NBUF],
            )
            cp.start()
            return cp

        def start_out(i):
            cp = pltpu.make_async_copy(
                bufs[i % _NBUF],
                out_hbm.at[pl.ds(base + i * _CHUNK, _CHUNK)],
                out_sems[i % _NBUF],
            )
            cp.start()
            return cp

        in_cps = [None] * _NCHUNK
        out_cps = [None] * _NCHUNK
        for b in range(_NBUF):
            in_cps[b] = start_in(b)
        for i in range(_NCHUNK):
            in_cps[i].wait()
            out_cps[i] = start_out(i)
            j = i + _NBUF
            if j < _NCHUNK:
                out_cps[i].wait()
                in_cps[j] = start_in(j)
        for i in range(_NCHUNK - _NBUF, _NCHUNK):
            out_cps[i].wait()

    return sc_copy


_sc_copy = _make_sc_copy()


@jax.jit
def kernel(pos_embd_weight):
    return _sc_copy(pos_embd_weight)[None]


def _tc_copy(x):
    def body(in_ref, out_ref):
        out_ref[...] = in_ref[...]

    return pl.pallas_call(
        body,
        out_shape=jax.ShapeDtypeStruct((_LENGTH, _N_EMBD), jnp.float32),
        grid=(16,),
        in_specs=[pl.BlockSpec((256, _N_EMBD), lambda i: (i, 0))],
        out_specs=pl.BlockSpec((256, _N_EMBD), lambda i: (i, 0)),
    )(x)


@jax.jit
def _kernel_tc(pos_embd_weight):
    return _tc_copy(pos_embd_weight)[None]

kernel = _kernel_tc
